# Initial kernel scaffold; baseline (speedup 1.0000x reference)
import jax
import jax.numpy as jnp
from jax.experimental import pallas as pl


def kernel(output, label):
    raise NotImplementedError("write your pallas kernel here")


# speedup vs baseline: 2.1172x; 2.1172x over previous
import functools
import math

import jax
import jax.numpy as jnp
from jax.experimental import pallas as pl
from jax.experimental.pallas import tpu as pltpu

_CORRECT_PROB = 0.99
_TFKD_ALPHA = 0.1
_TFKD_T = 20.0
_TFKD_MULT = 100.0
_OUTER_ALPHA = 0.1

_NEG = -1e30


def _consts(B, K):
    c = (1.0 - _CORRECT_PROB) / (K - 1)
    za = math.exp(_CORRECT_PROB / _TFKD_T)
    zb = math.exp(c / _TFKD_T)
    Z = za + (K - 1) * zb
    a = za / Z
    b = zb / Z
    C1 = -(1.0 - _TFKD_ALPHA) / B - _TFKD_ALPHA * _TFKD_MULT / (B * K) * (a - b)
    C2 = -_TFKD_ALPHA * _TFKD_MULT / (B * K) * b
    C3 = -_OUTER_ALPHA * c / B
    bias = _TFKD_ALPHA * _TFKD_MULT / K * (
        a * math.log(a) + (K - 1) * b * math.log(b))
    return C1, C2, C3, bias


def _body(x_ref, lab_ref, out_ref, work_ref, *, R, K, C1, C2, C3, bias):
    x = x_ref[...]
    lab = lab_ref[...]

    ii = jax.lax.broadcasted_iota(jnp.int32, (R, K), 1)
    xm = jnp.max(x, axis=1, keepdims=True)
    se = jnp.sum(jnp.exp(x - xm), axis=1, keepdims=True)
    lse = xm + jnp.log(se)
    rowsum = jnp.sum(x, axis=1, keepdims=True)
    xlab = jnp.sum(jnp.where(ii == lab, x, 0.0), axis=1, keepdims=True)
    logp_lab = xlab - lse
    S = rowsum - K * lse

    work_ref[...] = x
    il = jax.lax.broadcasted_iota(jnp.int32, (R, 128), 1)

    def step(r, top):
        w = work_ref[...]
        m = jnp.max(w, axis=1, keepdims=True)
        cand = jnp.where(w == m, ii, K)
        fi = jnp.min(cand, axis=1, keepdims=True)
        work_ref[...] = jnp.where(ii == fi, _NEG, w)
        return jnp.where(il == r, m, top)

    top = jax.lax.fori_loop(0, 100, step,
                            jnp.full((R, 128), _NEG, dtype=jnp.float32))

    acc = jnp.zeros((R, 1), dtype=jnp.float32)
    for w_ in range(19):
        lo = 5 * w_
        msk = (il >= lo) & (il < lo + 10)
        mw = jnp.max(jnp.where(msk, top, _NEG), axis=1, keepdims=True)
        sv = jnp.sum(jnp.where(msk, top, 0.0), axis=1, keepdims=True)
        ew = jnp.sum(jnp.where(msk, jnp.exp(top - mw), 0.0),
                     axis=1, keepdims=True)
        acc = acc + sv - 10.0 * (mw + jnp.log(ew))

    partial = jnp.sum(C1 * logp_lab + C2 * S + C3 * acc,
                      axis=(0, 1), keepdims=True)

    @pl.when(pl.program_id(0) == 0)
    def _():
        out_ref[...] = jnp.full((1, 1), bias, dtype=jnp.float32)

    out_ref[...] += partial


def kernel(output, label):
    B, K = output.shape
    R = 256
    C1, C2, C3, bias = _consts(B, K)
    lab2 = label.astype(jnp.int32).reshape(B, 1)
    body = functools.partial(_body, R=R, K=K, C1=C1, C2=C2, C3=C3, bias=bias)
    out = pl.pallas_call(
        body,
        grid=(B // R,),
        in_specs=[
            pl.BlockSpec((R, K), lambda i: (i, 0)),
            pl.BlockSpec((R, 1), lambda i: (i, 0)),
        ],
        out_specs=pl.BlockSpec((1, 1), lambda i: (0, 0)),
        out_shape=jax.ShapeDtypeStruct((1, 1), jnp.float32),
        scratch_shapes=[pltpu.VMEM((R, K), jnp.float32)],
    )(output, lab2)
    return out[0, 0]


# speedup vs baseline: 11.3454x; 5.3588x over previous
import functools
import math

import jax
import jax.numpy as jnp
from jax import lax
from jax.experimental import pallas as pl
from jax.experimental.pallas import tpu as pltpu
from jax.experimental.pallas import tpu_sc as plsc

_CORRECT_PROB = 0.99
_TFKD_ALPHA = 0.1
_TFKD_T = 20.0
_TFKD_MULT = 100.0
_OUTER_ALPHA = 0.1

_NEG = -1e30


def _consts(B, K):
    c = (1.0 - _CORRECT_PROB) / (K - 1)
    za = math.exp(_CORRECT_PROB / _TFKD_T)
    zb = math.exp(c / _TFKD_T)
    Z = za + (K - 1) * zb
    a = za / Z
    b = zb / Z
    C1 = -(1.0 - _TFKD_ALPHA) / B - _TFKD_ALPHA * _TFKD_MULT / (B * K) * (a - b)
    C2 = -_TFKD_ALPHA * _TFKD_MULT / (B * K) * b
    C3 = -_OUTER_ALPHA * c / B
    bias = _TFKD_ALPHA * _TFKD_MULT / K * (
        a * math.log(a) + (K - 1) * b * math.log(b))
    return C1, C2, C3, bias


def _vsort(v):
    k, _ = plsc.sort_key_val(v, v)
    return k


def _vrev(v):
    return lax.rev(v, dimensions=(0,))


def _rev_run(run):
    return [_vrev(v) for v in reversed(run)]


def _bitonic_local(run):
    l = len(run)
    d = l // 2
    while d >= 1:
        nr = list(run)
        for base in range(0, l, 2 * d):
            for i in range(base, base + d):
                a, b = run[i], run[i + d]
                nr[i] = jnp.minimum(a, b)
                nr[i + d] = jnp.maximum(a, b)
        run = nr
        d //= 2
    return [_vsort(v) for v in run]


def _merge_full(A, B):
    l = len(A)
    rB = _rev_run(B)
    lo = [jnp.minimum(A[i], rB[i]) for i in range(l)]
    hi = [jnp.maximum(A[i], rB[i]) for i in range(l)]
    return _bitonic_local(lo) + _bitonic_local(hi)


def _merge_top(A, B):
    l = len(A)
    rB = _rev_run(B)
    hi = [jnp.maximum(A[i], rB[i]) for i in range(l)]
    return _bitonic_local(hi)


def _top128(vregs):
    runs = [[_vsort(v)] for v in vregs]
    while len(runs) > 8:
        runs = [_merge_full(runs[i], runs[i + 1])
                for i in range(0, len(runs), 2)]
    while len(runs) > 1:
        runs = [_merge_top(runs[i], runs[i + 1])
                for i in range(0, len(runs), 2)]
    return runs[0]


def _sc_body(in_hbm, out_hbm, buf, obuf, *, K, rows_per, CH, NC):
    wid = lax.axis_index("s") * NC + lax.axis_index("c")
    nchunks = rows_per // CH
    nfull = K // 16
    ktail = K - nfull * 16

    def chunk_body(ch, carry):
        base_row = wid * rows_per + ch * CH
        pltpu.sync_copy(in_hbm.at[pl.ds(base_row * K, CH * K)],
                        buf.at[pl.ds(0, CH * K)])

        def row_body(r, carry2):
            off = r * K
            vregs = [buf[pl.ds(off + 16 * i, 16)] for i in range(nfull)]
            tail = buf[pl.ds(off + nfull * 16, 16)]
            lanes = lax.iota(jnp.int32, 16)
            vregs.append(jnp.where(lanes < ktail, tail,
                                   jnp.float32(_NEG)))
            vregs.append(jnp.full((16,), _NEG, jnp.float32))
            top = _top128(vregs)
            for j in range(8):
                obuf[pl.ds(r * 128 + 16 * j, 16)] = top[j]
            return carry2

        lax.fori_loop(0, CH, row_body, 0)
        pltpu.sync_copy(obuf, out_hbm.at[pl.ds(base_row * 128, CH * 128)])
        return carry

    lax.fori_loop(0, nchunks, chunk_body, 0)


def _sc_topk(xflat, B, K):
    info = plsc.get_sparse_core_info()
    NC, NS = info.num_cores, info.num_subcores
    NW = NC * NS
    rows_per = B // NW
    CH = 16
    mesh = plsc.VectorSubcoreMesh(core_axis_name="c", subcore_axis_name="s")
    body = functools.partial(_sc_body, K=K, rows_per=rows_per, CH=CH, NC=NC)
    fn = pl.kernel(
        body,
        mesh=mesh,
        compiler_params=pltpu.CompilerParams(needs_layout_passes=False),
        out_type=jax.ShapeDtypeStruct((B * 128,), jnp.float32),
        scratch_types=[
            pltpu.VMEM((CH * K + 16,), jnp.float32),
            pltpu.VMEM((CH * 128,), jnp.float32),
        ],
    )
    return fn(xflat)


def _stats_body(x_ref, lab_ref, out_ref, *, R, K, C1, C2, bias):
    x = x_ref[...]
    lab = lab_ref[...]
    ii = lax.broadcasted_iota(jnp.int32, (R, K), 1)
    xm = jnp.max(x, axis=1, keepdims=True)
    se = jnp.sum(jnp.exp(x - xm), axis=1, keepdims=True)
    lse = xm + jnp.log(se)
    rowsum = jnp.sum(x, axis=1, keepdims=True)
    xlab = jnp.sum(jnp.where(ii == lab, x, 0.0), axis=1, keepdims=True)
    logp_lab = xlab - lse
    S = rowsum - K * lse
    partial = jnp.sum(C1 * logp_lab + C2 * S, axis=(0, 1), keepdims=True)

    @pl.when(pl.program_id(0) == 0)
    def _():
        out_ref[...] = jnp.full((1, 1), bias, dtype=jnp.float32)

    out_ref[...] += partial


def _win_body(top_ref, out_ref, *, R, C3):
    top = top_ref[...]
    il = lax.broadcasted_iota(jnp.int32, (R, 128), 1)
    acc = jnp.zeros((R, 1), dtype=jnp.float32)
    for w in range(19):
        lo = 118 - 5 * w
        msk = (il >= lo) & (il < lo + 10)
        mw = jnp.max(jnp.where(msk, top, _NEG), axis=1, keepdims=True)
        sv = jnp.sum(jnp.where(msk, top, 0.0), axis=1, keepdims=True)
        ew = jnp.sum(jnp.where(msk, jnp.exp(top - mw), 0.0),
                     axis=1, keepdims=True)
        acc = acc + sv - 10.0 * (mw + jnp.log(ew))
    partial = jnp.sum(C3 * acc, axis=(0, 1), keepdims=True)

    @pl.when(pl.program_id(0) == 0)
    def _():
        out_ref[...] = jnp.zeros((1, 1), dtype=jnp.float32)

    out_ref[...] += partial


def kernel(output, label):
    B, K = output.shape
    C1, C2, C3, bias = _consts(B, K)

    top = _sc_topk(output.reshape(-1), B, K).reshape(B, 128)

    R = 256
    lab2 = label.astype(jnp.int32).reshape(B, 1)
    stats = pl.pallas_call(
        functools.partial(_stats_body, R=R, K=K, C1=C1, C2=C2, bias=bias),
        grid=(B // R,),
        in_specs=[
            pl.BlockSpec((R, K), lambda i: (i, 0)),
            pl.BlockSpec((R, 1), lambda i: (i, 0)),
        ],
        out_specs=pl.BlockSpec((1, 1), lambda i: (0, 0)),
        out_shape=jax.ShapeDtypeStruct((1, 1), jnp.float32),
    )(output, lab2)

    R2 = 512
    win = pl.pallas_call(
        functools.partial(_win_body, R=R2, C3=C3),
        grid=(B // R2,),
        in_specs=[pl.BlockSpec((R2, 128), lambda i: (i, 0))],
        out_specs=pl.BlockSpec((1, 1), lambda i: (0, 0)),
        out_shape=jax.ShapeDtypeStruct((1, 1), jnp.float32),
    )(top)

    return stats[0, 0] + win[0, 0]
